# Initial kernel scaffold; baseline (speedup 1.0000x reference)
import jax
import jax.numpy as jnp
from jax.experimental import pallas as pl


def kernel(x, edge_index, W1, b1, W2, b2):
    raise NotImplementedError("write your pallas kernel here")


# speedup vs baseline: 17.1459x; 17.1459x over previous
import functools

import jax
import jax.numpy as jnp
from jax import lax
from jax.experimental import pallas as pl
from jax.experimental.pallas import tpu as pltpu
from jax.experimental.pallas import tpu_sc as plsc

N = 10000
E = 320000
NC = 2
NS = 16
NW = NC * NS
KB = 128
NBLK = E // KB
NB_LO = NBLK // NW
NB_REM = NBLK - NB_LO * NW

N_PAD = 10240
ZROWS = 128

_MESH = plsc.VectorSubcoreMesh(
    core_axis_name="c", subcore_axis_name="s", num_cores=NC, num_subcores=NS)


def _worker_id():
  c = lax.axis_index("c")
  s = lax.axis_index("s")
  return c * NS + s, c, s


def _num_blocks(wid):
  return NB_LO + jnp.where(wid < NB_REM, 1, 0)


@functools.partial(
    pl.kernel,
    out_type=jax.ShapeDtypeStruct((NC, N_PAD), jnp.float32),
    mesh=_MESH,
    scratch_types=[
        pltpu.VMEM((KB,), jnp.int32),
        pltpu.VMEM((KB,), jnp.float32),
        pltpu.VMEM((640,), jnp.float32),
        pltpu.VMEM_SHARED((N_PAD,), jnp.float32),
        pltpu.SemaphoreType.DMA,
    ],
)
def _sc_degree(dst_hbm, out_hbm, idx_v, ones_v, zb_v, deg_sh, sem):
  wid, c, s = _worker_id()

  for i in range(KB // 16):
    ones_v[pl.ds(i * 16, 16)] = jnp.full((16,), 1.0, jnp.float32)
  for i in range(640 // 16):
    zb_v[pl.ds(i * 16, 16)] = jnp.zeros((16,), jnp.float32)

  pltpu.sync_copy(zb_v, deg_sh.at[pl.ds(s * 640, 640)])
  plsc.subcore_barrier()

  def body(j, carry):
    off = pl.multiple_of((wid + j * NW) * KB, KB)
    pltpu.sync_copy(dst_hbm.at[pl.ds(off, KB)], idx_v)
    pltpu.sync_copy(ones_v, deg_sh.at[idx_v], add=True)
    return carry

  lax.fori_loop(0, _num_blocks(wid), body, 0)
  plsc.subcore_barrier()

  pltpu.sync_copy(deg_sh.at[pl.ds(s * 640, 640)], zb_v)
  pltpu.sync_copy(zb_v, out_hbm.at[c, pl.ds(s * 640, 640)])


def _make_sc_prop(d):
  @functools.partial(
      pl.kernel,
      out_type=jax.ShapeDtypeStruct((NC, N_PAD, d), jnp.float32),
      mesh=_MESH,
      compiler_params=pltpu.CompilerParams(use_tc_tiling_on_sc=(d % 128 == 0)),
      scratch_types=[
          pltpu.VMEM((KB,), jnp.int32),
          pltpu.VMEM((KB,), jnp.int32),
          pltpu.VMEM((KB, d), jnp.float32),
          pltpu.VMEM((ZROWS, d), jnp.float32),
          pltpu.VMEM_SHARED((N_PAD, d), jnp.float32),
          pltpu.SemaphoreType.DMA,
      ],
  )
  def prop(g_hbm, src_hbm, dst_hbm, out_hbm, is_v, id_v, rows_v, zb_v,
           acc_sh, sem):
    wid, c, s = _worker_id()

    nlane = d // 16
    def zinit(i, carry):
      r = i // nlane
      col = (i % nlane) * 16
      zb_v[r, pl.ds(col, 16)] = jnp.zeros((16,), jnp.float32)
      return carry
    lax.fori_loop(0, ZROWS * nlane, zinit, 0)
    for j in range(5):
      pltpu.sync_copy(zb_v, acc_sh.at[pl.ds(s * 640 + j * ZROWS, ZROWS)])
    plsc.subcore_barrier()

    def body(j, carry):
      off = pl.multiple_of((wid + j * NW) * KB, KB)
      pltpu.sync_copy(src_hbm.at[pl.ds(off, KB)], is_v)
      pltpu.sync_copy(dst_hbm.at[pl.ds(off, KB)], id_v)
      pltpu.async_copy(g_hbm.at[is_v], rows_v, sem).wait()
      pltpu.sync_copy(rows_v, acc_sh.at[id_v], add=True)
      return carry

    lax.fori_loop(0, _num_blocks(wid), body, 0)
    plsc.subcore_barrier()

    for j in range(5):
      r0 = s * 640 + j * ZROWS
      pltpu.sync_copy(acc_sh.at[pl.ds(r0, ZROWS)], zb_v)
      pltpu.sync_copy(zb_v, out_hbm.at[c, pl.ds(r0, ZROWS)])

  return prop


_sc_prop_128 = _make_sc_prop(128)
_sc_prop_48 = _make_sc_prop(48)


_RB = 2000


def _tc1_body(x_ref, w1_ref, d0_ref, d1_ref, g1_ref, dinv_ref):
  deg = d0_ref[...] + d1_ref[...] + 1.0
  dinv = lax.rsqrt(jnp.maximum(deg, 1.0))
  h = lax.dot_general(x_ref[...], w1_ref[...], (((1,), (1,)), ((), ())),
                      preferred_element_type=jnp.float32)
  g1_ref[...] = dinv * h
  dinv_ref[...] = dinv


def _tc1(x, w1, d0, d1):
  return pl.pallas_call(
      _tc1_body,
      grid=(N // _RB,),
      in_specs=[
          pl.BlockSpec((_RB, 128), lambda i: (i, 0)),
          pl.BlockSpec((128, 128), lambda i: (0, 0)),
          pl.BlockSpec((_RB, 1), lambda i: (i, 0)),
          pl.BlockSpec((_RB, 1), lambda i: (i, 0)),
      ],
      out_specs=[
          pl.BlockSpec((_RB, 128), lambda i: (i, 0)),
          pl.BlockSpec((_RB, 1), lambda i: (i, 0)),
      ],
      out_shape=[
          jax.ShapeDtypeStruct((N, 128), jnp.float32),
          jax.ShapeDtypeStruct((N, 1), jnp.float32),
      ],
  )(x, w1, d0, d1)


def _tc2_body(s0_ref, s1_ref, g1_ref, dinv_ref, b1_ref, w2_ref, g2_ref):
  dinv = dinv_ref[...]
  t = (s0_ref[...] + s1_ref[...] + g1_ref[...]) * dinv + b1_ref[...]
  o1 = jnp.maximum(t, 0.0)
  h2 = lax.dot_general(o1, w2_ref[...], (((1,), (1,)), ((), ())),
                       preferred_element_type=jnp.float32)
  g2_ref[...] = dinv * h2


def _tc2(s0, s1, g1, dinv, b1, w2p):
  return pl.pallas_call(
      _tc2_body,
      grid=(N // _RB,),
      in_specs=[
          pl.BlockSpec((_RB, 128), lambda i: (i, 0)),
          pl.BlockSpec((_RB, 128), lambda i: (i, 0)),
          pl.BlockSpec((_RB, 128), lambda i: (i, 0)),
          pl.BlockSpec((_RB, 1), lambda i: (i, 0)),
          pl.BlockSpec((1, 128), lambda i: (0, 0)),
          pl.BlockSpec((48, 128), lambda i: (0, 0)),
      ],
      out_specs=pl.BlockSpec((_RB, 48), lambda i: (i, 0)),
      out_shape=jax.ShapeDtypeStruct((N, 48), jnp.float32),
  )(s0, s1, g1, dinv, b1, w2p)


def _tc3_body(s0_ref, s1_ref, g2_ref, dinv_ref, b2_ref, out_ref):
  out_ref[...] = ((s0_ref[...] + s1_ref[...] + g2_ref[...]) * dinv_ref[...]
                  + b2_ref[...])


def _tc3(s0, s1, g2, dinv, b2p):
  return pl.pallas_call(
      _tc3_body,
      grid=(N // _RB,),
      in_specs=[
          pl.BlockSpec((_RB, 48), lambda i: (i, 0)),
          pl.BlockSpec((_RB, 48), lambda i: (i, 0)),
          pl.BlockSpec((_RB, 48), lambda i: (i, 0)),
          pl.BlockSpec((_RB, 1), lambda i: (i, 0)),
          pl.BlockSpec((1, 48), lambda i: (0, 0)),
      ],
      out_specs=pl.BlockSpec((_RB, 48), lambda i: (i, 0)),
      out_shape=jax.ShapeDtypeStruct((N, 48), jnp.float32),
  )(s0, s1, g2, dinv, b2p)


def kernel(x, edge_index, W1, b1, W2, b2):
  src = edge_index[0].astype(jnp.int32)
  dst = edge_index[1].astype(jnp.int32)

  degp = _sc_degree(dst)
  d0 = degp[0, :N, None]
  d1 = degp[1, :N, None]

  g1, dinv = _tc1(x, W1, d0, d1)
  s1 = _sc_prop_128(g1, src, dst)[:, :N]

  w2p = jnp.zeros((48, 128), jnp.float32).at[:40].set(W2)
  b1r = b1.reshape(1, 128)
  b2p = jnp.zeros((1, 48), jnp.float32).at[0, :40].set(b2)

  g2 = _tc2(s1[0], s1[1], g1, dinv, b1r, w2p)
  s2 = _sc_prop_48(g2, src, dst)[:, :N]
  out = _tc3(s2[0], s2[1], g2, dinv, b2p)
  return out[:, :40]
